# baseline (speedup 1.0000x reference)
import functools

import jax
import jax.numpy as jnp
from jax import lax
from jax.experimental import pallas as pl
from jax.experimental.pallas import tpu as pltpu
from jax.experimental.pallas import tpu_sc as plsc

DIM = 1024
HIDDEN = 512
E = 64
TOKENS = 2048
CAP = 36
PAD = 40
SLOT_LANES = 128
CHUNK = 256
NCHUNK = TOKENS // CHUNK
AUX_W = 0.01


def _router_body(x_ref, rwt_ref, rb_ref, xidx_ref, gates_ref, cmb_ref, aux_ref):
    f32 = jnp.float32
    xf = x_ref[...]
    logits = jnp.dot(xf, rwt_ref[...], preferred_element_type=f32) + rb_ref[...]
    m = jnp.max(logits, axis=1, keepdims=True)
    ex = jnp.exp(logits - m)
    s = jnp.sum(ex, axis=1, keepdims=True)
    probs = ex / s

    maxp = jnp.max(probs, axis=1, keepdims=True)
    gcol = maxp / (maxp + 1e-9)
    iot_e = lax.broadcasted_iota(jnp.int32, (TOKENS, E), 1).astype(f32)
    ecol = jnp.min(jnp.where(probs == maxp, iot_e, float(E)), axis=1, keepdims=True)

    diag = (lax.broadcasted_iota(jnp.int32, (CHUNK, CHUNK), 0)
            == lax.broadcasted_iota(jnp.int32, (CHUNK, CHUNK), 1)).astype(f32)
    g_rows, e_rows = [], []
    for c in range(NCHUNK):
        sl = slice(c * CHUNK, (c + 1) * CHUNK)
        g_rows.append(jnp.sum(diag * gcol[sl], axis=0, keepdims=True))
        e_rows.append(jnp.sum(diag * ecol[sl], axis=0, keepdims=True))
    g_row = jnp.concatenate(g_rows, axis=1)
    e_row = jnp.concatenate(e_rows, axis=1)
    t_row = lax.broadcasted_iota(jnp.int32, (1, TOKENS), 1).astype(f32)

    ranks = []
    for c in range(NCHUNK):
        sl = slice(c * CHUNK, (c + 1) * CHUNK)
        gseg = gcol[sl]
        eseg = ecol[sl]
        tcol = lax.broadcasted_iota(jnp.int32, (CHUNK, 1), 0).astype(f32) + float(c * CHUNK)
        same = e_row == eseg
        better = (g_row > gseg) | ((g_row == gseg) & (t_row < tcol))
        ranks.append(jnp.sum(jnp.where(same & better, 1.0, 0.0), axis=1,
                             keepdims=True))
    rank = jnp.concatenate(ranks, axis=0)
    kept = rank < float(CAP)

    p_t = jnp.where(lax.broadcasted_iota(jnp.int32, (E, TOKENS), 0).astype(f32) == e_row, 1.0, 0.0)
    iota_s = lax.broadcasted_iota(jnp.int32, (TOKENS, SLOT_LANES), 1).astype(f32)
    r_mat = jnp.where((rank == iota_s) & kept, 1.0, 0.0)
    tokcol = lax.broadcasted_iota(jnp.int32, (TOKENS, 1), 0).astype(f32)
    table = jnp.dot(p_t, r_mat * tokcol, preferred_element_type=f32)
    gates_tab = jnp.dot(p_t, r_mat * gcol, preferred_element_type=f32)
    xidx_ref[...] = table.astype(jnp.int32)
    gates_ref[...] = gates_tab
    cmb_ref[...] = jnp.where(kept, ecol * float(PAD) + rank,
                             float(CAP)).astype(jnp.int32)

    p_col = jnp.where(iot_e == ecol, 1.0, 0.0)
    counts = jnp.sum(p_col, axis=0, keepdims=True)
    meanp = jnp.sum(probs, axis=0, keepdims=True)
    aux_ref[...] = jnp.sum(counts * meanp, axis=1, keepdims=True) * (
        float(E) * AUX_W / float(TOKENS * TOKENS))


def _router_call(xflat, rwt, rb_row):
    return pl.pallas_call(
        _router_body,
        out_shape=[
            jax.ShapeDtypeStruct((E, SLOT_LANES), jnp.int32),
            jax.ShapeDtypeStruct((E, SLOT_LANES), jnp.float32),
            jax.ShapeDtypeStruct((TOKENS, 1), jnp.int32),
            jax.ShapeDtypeStruct((1, 1), jnp.float32),
        ],
    )(xflat, rwt, rb_row)


def _expert_body(xg_ref, g_ref, w1_ref, b1_ref, w2_ref, b2_ref, y_ref):
    f32 = jnp.float32
    xin = xg_ref[0]
    h = lax.dot_general(xin, w1_ref[0], (((1,), (1,)), ((), ())),
                        preferred_element_type=f32)
    h = h + b1_ref[0]
    h = h * 0.5 * (1.0 + lax.erf(h * (2.0 ** -0.5)))
    y = lax.dot_general(h, w2_ref[0], (((1,), (1,)), ((), ())),
                        preferred_element_type=f32)
    y_ref[0] = (y + b2_ref[0]) * g_ref[0]


def _experts_call(xg, gates3, w1, b1, w2, b2):
    return pl.pallas_call(
        _expert_body,
        grid=(E,),
        in_specs=[
            pl.BlockSpec((1, PAD, DIM), lambda e: (e, 0, 0)),
            pl.BlockSpec((1, PAD, 1), lambda e: (e, 0, 0)),
            pl.BlockSpec((1, HIDDEN, DIM), lambda e: (e, 0, 0)),
            pl.BlockSpec((1, 1, HIDDEN), lambda e: (e, 0, 0)),
            pl.BlockSpec((1, DIM, HIDDEN), lambda e: (e, 0, 0)),
            pl.BlockSpec((1, 1, DIM), lambda e: (e, 0, 0)),
        ],
        out_specs=pl.BlockSpec((1, PAD, DIM), lambda e: (e, 0, 0)),
        out_shape=jax.ShapeDtypeStruct((E, PAD, DIM), jnp.float32),
        compiler_params=pltpu.CompilerParams(
            dimension_semantics=("parallel",)),
    )(xg, gates3, w1, b1, w2, b2)


def _sc_gather(table, idx, out_rows, d):
    info = plsc.get_sparse_core_info()
    nc, ns = info.num_cores, info.num_subcores
    nw = nc * ns
    bpw = out_rows // nw
    ch = 16
    nchunk = bpw // ch
    mesh = plsc.VectorSubcoreMesh(core_axis_name="c", subcore_axis_name="s")

    @functools.partial(
        pl.kernel, mesh=mesh,
        out_type=jax.ShapeDtypeStruct((out_rows, d), jnp.float32),
        scratch_types=[
            pltpu.VMEM((bpw,), jnp.int32),
            pltpu.VMEM((bpw, d), jnp.float32),
            pltpu.SemaphoreType.DMA,
            pltpu.SemaphoreType.DMA,
        ],
    )
    def k(table_hbm, idx_hbm, out_hbm, idx_v, rows_v, gsem, wsem):
        wid = lax.axis_index("s") * nc + lax.axis_index("c")
        base = wid * bpw
        pltpu.sync_copy(idx_hbm.at[pl.ds(base, bpw)], idx_v)
        gcps = []
        for c in range(nchunk):
            gcps.append(pltpu.async_copy(
                table_hbm.at[idx_v.at[pl.ds(c * ch, ch)]],
                rows_v.at[pl.ds(c * ch, ch)], gsem))
        wcps = []
        for c in range(nchunk):
            gcps[c].wait()
            wcps.append(pltpu.async_copy(
                rows_v.at[pl.ds(c * ch, ch)],
                out_hbm.at[pl.ds(base + c * ch, ch)], wsem))
        for c in range(nchunk):
            wcps[c].wait()

    return k(table, idx)


def kernel(x, router_w, router_b, w1, b1, w2, b2):
    b, n, d = x.shape
    flat = x.reshape(TOKENS, DIM)
    xidx_i, gates_tab, cmb, aux = _router_call(
        flat, router_w.T, router_b.reshape(1, E))
    xidx = xidx_i[:, :PAD].reshape(E * PAD)
    gates3 = gates_tab[:, :PAD].reshape(E, PAD, 1)
    xg = _sc_gather(flat, xidx, E * PAD, DIM)
    y_all = _experts_call(xg.reshape(E, PAD, DIM), gates3, w1,
                          b1.reshape(E, 1, HIDDEN), w2, b2.reshape(E, 1, DIM))
    out_flat = _sc_gather(y_all.reshape(E * PAD, DIM), cmb.reshape(TOKENS),
                          TOKENS, DIM)
    return out_flat.reshape(b, n, d), aux[0, 0]


# speedup vs baseline: 1.2450x; 1.2450x over previous
import functools

import jax
import jax.numpy as jnp
from jax import lax
from jax.experimental import pallas as pl
from jax.experimental.pallas import tpu as pltpu
from jax.experimental.pallas import tpu_sc as plsc

DIM = 1024
HIDDEN = 512
E = 64
TOKENS = 2048
CAP = 36
PAD = 40
SLOT_LANES = 128
CHUNK = 256
NCHUNK = TOKENS // CHUNK
AUX_W = 0.01


def _router_body(x_ref, rwt_ref, rb_ref, xidx_ref, gates_ref, cmb_ref, aux_ref):
    f32 = jnp.float32
    xf = x_ref[...]
    logits = jnp.dot(xf, rwt_ref[...], preferred_element_type=f32) + rb_ref[...]
    m = jnp.max(logits, axis=1, keepdims=True)
    ex = jnp.exp(logits - m)
    s = jnp.sum(ex, axis=1, keepdims=True)
    probs = ex / s

    maxp = jnp.max(probs, axis=1, keepdims=True)
    gcol = maxp / (maxp + 1e-9)
    iot_e = lax.broadcasted_iota(jnp.int32, (TOKENS, E), 1).astype(f32)
    ecol = jnp.min(jnp.where(probs == maxp, iot_e, float(E)), axis=1, keepdims=True)

    diag = (lax.broadcasted_iota(jnp.int32, (CHUNK, CHUNK), 0)
            == lax.broadcasted_iota(jnp.int32, (CHUNK, CHUNK), 1)).astype(f32)
    g_rows, e_rows = [], []
    for c in range(NCHUNK):
        sl = slice(c * CHUNK, (c + 1) * CHUNK)
        g_rows.append(jnp.sum(diag * gcol[sl], axis=0, keepdims=True))
        e_rows.append(jnp.sum(diag * ecol[sl], axis=0, keepdims=True))
    g_row = jnp.concatenate(g_rows, axis=1)
    e_row = jnp.concatenate(e_rows, axis=1)
    t_row = lax.broadcasted_iota(jnp.int32, (1, TOKENS), 1).astype(f32)

    ranks = []
    for c in range(NCHUNK):
        sl = slice(c * CHUNK, (c + 1) * CHUNK)
        gseg = gcol[sl]
        eseg = ecol[sl]
        tcol = lax.broadcasted_iota(jnp.int32, (CHUNK, 1), 0).astype(f32) + float(c * CHUNK)
        same = e_row == eseg
        better = (g_row > gseg) | ((g_row == gseg) & (t_row < tcol))
        ranks.append(jnp.sum(jnp.where(same & better, 1.0, 0.0), axis=1,
                             keepdims=True))
    rank = jnp.concatenate(ranks, axis=0)
    kept = rank < float(CAP)

    p_t = jnp.where(lax.broadcasted_iota(jnp.int32, (E, TOKENS), 0).astype(f32) == e_row, 1.0, 0.0)
    iota_s = lax.broadcasted_iota(jnp.int32, (TOKENS, SLOT_LANES), 1).astype(f32)
    r_mat = jnp.where((rank == iota_s) & kept, 1.0, 0.0)
    tokcol = lax.broadcasted_iota(jnp.int32, (TOKENS, 1), 0).astype(f32)
    gates_tab = jnp.dot(p_t, r_mat * gcol, preferred_element_type=f32)
    gates_ref[...] = gates_tab
    table = jnp.dot(p_t, r_mat * tokcol, preferred_element_type=f32)
    spread = (lax.broadcasted_iota(jnp.int32, (E, SLOT_LANES), 0) * PAD
              + lax.broadcasted_iota(jnp.int32, (E, SLOT_LANES), 1)) & (TOKENS - 1)
    xidx_ref[...] = jnp.where(gates_tab > 0.0, table.astype(jnp.int32), spread)
    t_i = lax.broadcasted_iota(jnp.int32, (TOKENS, 1), 0)
    dump = (t_i & (E - 1)) * PAD + CAP + ((t_i >> 6) & 3)
    cmb_ref[...] = jnp.where(kept, (ecol * float(PAD) + rank).astype(jnp.int32),
                             dump)

    p_col = jnp.where(iot_e == ecol, 1.0, 0.0)
    counts = jnp.sum(p_col, axis=0, keepdims=True)
    meanp = jnp.sum(probs, axis=0, keepdims=True)
    aux_ref[...] = jnp.sum(counts * meanp, axis=1, keepdims=True) * (
        float(E) * AUX_W / float(TOKENS * TOKENS))


def _router_call(xflat, rwt, rb_row):
    return pl.pallas_call(
        _router_body,
        out_shape=[
            jax.ShapeDtypeStruct((E, SLOT_LANES), jnp.int32),
            jax.ShapeDtypeStruct((E, SLOT_LANES), jnp.float32),
            jax.ShapeDtypeStruct((TOKENS, 1), jnp.int32),
            jax.ShapeDtypeStruct((1, 1), jnp.float32),
        ],
    )(xflat, rwt, rb_row)


def _expert_body(xg_ref, g_ref, w1_ref, b1_ref, w2_ref, b2_ref, y_ref):
    f32 = jnp.float32
    xin = xg_ref[0]
    h = lax.dot_general(xin, w1_ref[0], (((1,), (1,)), ((), ())),
                        preferred_element_type=f32)
    h = h + b1_ref[0]
    h = h * 0.5 * (1.0 + lax.erf(h * (2.0 ** -0.5)))
    y = lax.dot_general(h, w2_ref[0], (((1,), (1,)), ((), ())),
                        preferred_element_type=f32)
    y_ref[0] = (y + b2_ref[0]) * g_ref[0]


def _experts_call(xg, gates3, w1, b1, w2, b2):
    return pl.pallas_call(
        _expert_body,
        grid=(E,),
        in_specs=[
            pl.BlockSpec((1, PAD, DIM), lambda e: (e, 0, 0)),
            pl.BlockSpec((1, PAD, 1), lambda e: (e, 0, 0)),
            pl.BlockSpec((1, HIDDEN, DIM), lambda e: (e, 0, 0)),
            pl.BlockSpec((1, 1, HIDDEN), lambda e: (e, 0, 0)),
            pl.BlockSpec((1, DIM, HIDDEN), lambda e: (e, 0, 0)),
            pl.BlockSpec((1, 1, DIM), lambda e: (e, 0, 0)),
        ],
        out_specs=pl.BlockSpec((1, PAD, DIM), lambda e: (e, 0, 0)),
        out_shape=jax.ShapeDtypeStruct((E, PAD, DIM), jnp.float32),
        compiler_params=pltpu.CompilerParams(
            dimension_semantics=("parallel",)),
    )(xg, gates3, w1, b1, w2, b2)


def _sc_gather(table, idx, out_rows, d):
    info = plsc.get_sparse_core_info()
    nc, ns = info.num_cores, info.num_subcores
    nw = nc * ns
    bpw = out_rows // nw
    mesh = plsc.VectorSubcoreMesh(core_axis_name="c", subcore_axis_name="s")

    @functools.partial(
        pl.kernel, mesh=mesh,
        out_type=jax.ShapeDtypeStruct((out_rows, d), jnp.float32),
        scratch_types=[
            pltpu.VMEM((bpw,), jnp.int32),
            pltpu.VMEM((bpw, d), jnp.float32),
            pltpu.SemaphoreType.DMA,
        ],
    )
    def k(table_hbm, idx_hbm, out_hbm, idx_v, rows_v, sem):
        wid = lax.axis_index("s") * nc + lax.axis_index("c")
        base = wid * bpw
        pltpu.sync_copy(idx_hbm.at[pl.ds(base, bpw)], idx_v)
        pltpu.async_copy(table_hbm.at[idx_v], rows_v, sem).wait()
        pltpu.sync_copy(rows_v, out_hbm.at[pl.ds(base, bpw)])

    return k(table, idx)


def kernel(x, router_w, router_b, w1, b1, w2, b2):
    b, n, d = x.shape
    flat = x.reshape(TOKENS, DIM)
    xidx_i, gates_tab, cmb, aux = _router_call(
        flat, router_w.T, router_b.reshape(1, E))
    gates3 = gates_tab[:, :PAD].reshape(E, PAD, 1)
    cmb_flat = cmb.reshape(TOKENS)
    xidx = xidx_i[:, :PAD].reshape(E * PAD)
    xg = _sc_gather(flat, xidx, E * PAD, DIM)
    y_all = _experts_call(xg.reshape(E, PAD, DIM), gates3, w1,
                          b1.reshape(E, 1, HIDDEN), w2, b2.reshape(E, 1, DIM))
    out_flat = _sc_gather(y_all.reshape(E * PAD, DIM), cmb_flat, TOKENS, DIM)
    return out_flat.reshape(b, n, d), aux[0, 0]
